# baseline (speedup 1.0000x reference)
import functools

import jax
import jax.numpy as jnp
from jax import lax
from jax.experimental import pallas as pl
from jax.experimental.pallas import tpu as pltpu
from jax.experimental.pallas import tpu_sc as plsc

D_MODEL = 768
NUM_ROWS = 6
BATCH = 16384

_info = plsc.get_sparse_core_info()
NUM_CORES = _info.num_cores
NUM_SUBCORES = _info.num_subcores
NUM_WORKERS = NUM_CORES * NUM_SUBCORES
B_PER_W = BATCH // NUM_WORKERS
P_PER_W = B_PER_W // 2
LANES = 16
N_GROUPS = P_PER_W // LANES
NUM_PAIRS = NUM_ROWS * NUM_ROWS
N_REPLICAS = 8
TILES_PER_REPLICA = NUM_WORKERS // N_REPLICAS

_mesh = plsc.VectorSubcoreMesh(core_axis_name="c", subcore_axis_name="s")


@functools.partial(
    pl.kernel,
    mesh=_mesh,
    out_type=jax.ShapeDtypeStruct((BATCH * D_MODEL,), jnp.float32),
    scratch_types=[
        pltpu.VMEM((P_PER_W,), jnp.int32),
        pltpu.VMEM((NUM_PAIRS * 2 * D_MODEL,), jnp.float32),
        pltpu.SemaphoreType.DMA,
    ],
)
def _emb_kernel(idx_hbm, pairs_hbm, out_hbm, idx_v, pt_v, wsem):
    cid = lax.axis_index("c")
    sid = lax.axis_index("s")
    wid = sid * NUM_CORES + cid
    PAIR_W = 2 * D_MODEL
    base = wid * B_PER_W * D_MODEL
    rid = lax.div(wid, TILES_PER_REPLICA)

    pltpu.sync_copy(pairs_hbm.at[rid], pt_v)
    pltpu.sync_copy(idx_hbm.at[wid], idx_v)

    def group_body(g, _):
        v = idx_v[pl.ds(g * LANES, LANES)]
        b = base + g * (LANES * PAIR_W)
        for l in range(LANES):
            e = v[l]
            pltpu.make_async_copy(
                pt_v.at[pl.ds(e * PAIR_W, PAIR_W)],
                out_hbm.at[pl.ds(b + l * PAIR_W, PAIR_W)],
                wsem,
            ).start()
        return 0

    lax.fori_loop(0, N_GROUPS, group_body, 0)

    def dbody(c, _):
        pltpu.make_async_copy(
            pt_v.at[pl.ds(0, PAIR_W)], out_hbm.at[pl.ds(base, PAIR_W)], wsem
        ).wait()
        return 0

    lax.fori_loop(0, P_PER_W, dbody, 0)


def kernel(emotion_id, table):
    if emotion_id.ndim > 1:
        emotion_id = emotion_id.reshape(-1)
    idx = emotion_id.astype(jnp.int32).reshape(NUM_WORKERS, P_PER_W, 2)
    pair_idx = idx[:, :, 0] * NUM_ROWS + idx[:, :, 1]
    first = jnp.broadcast_to(
        table[None, :, None, None, :],
        (N_REPLICAS, NUM_ROWS, NUM_ROWS, 1, D_MODEL),
    )
    second = jnp.broadcast_to(
        table[None, None, :, None, :],
        (N_REPLICAS, NUM_ROWS, NUM_ROWS, 1, D_MODEL),
    )
    pair_table = jnp.concatenate([first, second], axis=3)
    pair_table = pair_table.reshape(N_REPLICAS, NUM_PAIRS * 2 * D_MODEL)
    out = _emb_kernel(pair_idx, pair_table)
    return out.reshape(BATCH, D_MODEL)


# speedup vs baseline: 2.6849x; 2.6849x over previous
import functools

import jax
import jax.numpy as jnp
from jax import lax
from jax.experimental import pallas as pl
from jax.experimental.pallas import tpu as pltpu
from jax.experimental.pallas import tpu_sc as plsc

D_MODEL = 768
NUM_ROWS = 6
BATCH = 16384

_info = plsc.get_sparse_core_info()
NUM_CORES = _info.num_cores
NUM_SUBCORES = _info.num_subcores
NUM_WORKERS = NUM_CORES * NUM_SUBCORES
B_PER_W = BATCH // NUM_WORKERS
LANES = 16
N_GROUPS = B_PER_W // LANES

_mesh = plsc.VectorSubcoreMesh(core_axis_name="c", subcore_axis_name="s")


@functools.partial(
    pl.kernel,
    mesh=_mesh,
    out_type=jax.ShapeDtypeStruct((BATCH, D_MODEL), jnp.float32),
    scratch_types=[
        pltpu.VMEM((B_PER_W,), jnp.int32),
        pltpu.VMEM((NUM_ROWS, D_MODEL), jnp.float32),
        pltpu.SemaphoreType.DMA,
        pltpu.SemaphoreType.DMA,
        pltpu.SemaphoreType.DMA,
        pltpu.SemaphoreType.DMA,
    ],
)
def _emb_kernel(idx_hbm, table_hbm, out_hbm, idx_v, table_v,
                w0, w1, w2, w3):
    cid = lax.axis_index("c")
    sid = lax.axis_index("s")
    wid = sid * NUM_CORES + cid
    base = wid * B_PER_W

    pltpu.sync_copy(table_hbm, table_v)
    pltpu.sync_copy(idx_hbm.at[wid], idx_v)

    wsems = (w0, w1, w2, w3)

    def group_body(g, _):
        v = idx_v[pl.ds(g * LANES, LANES)]
        b = base + g * LANES
        for l in range(LANES):
            e = v[l]
            pltpu.make_async_copy(
                table_v.at[e], out_hbm.at[b + l], wsems[l % 4]
            ).start()
        return 0

    lax.fori_loop(0, N_GROUPS, group_body, 0)

    for q in range(4):
        pltpu.make_async_copy(
            out_hbm.at[pl.ds(base, B_PER_W // 4)],
            out_hbm.at[pl.ds(base, B_PER_W // 4)],
            wsems[q],
        ).wait()


def kernel(emotion_id, table):
    if emotion_id.ndim > 1:
        emotion_id = emotion_id.reshape(-1)
    idx = emotion_id.astype(jnp.int32).reshape(NUM_WORKERS, B_PER_W)
    return _emb_kernel(idx, table)
